# baseline (speedup 1.0000x reference)
import functools

import jax
import jax.numpy as jnp
from jax import lax
from jax.experimental import pallas as pl
from jax.experimental.pallas import tpu as pltpu
from jax.experimental.pallas import tpu_sc as plsc

EPS = 1e-8
TAU = 1.5
LC, LS, LA = 0.1, 0.05, 0.01
K = 8
NEG = -1e30
BIG = 2 ** 30
LOG2 = 0.6931471805599453

def _gbar_body(g_ref, gbar_ref, entg_ref, tab_ref):
    g = g_ref[...]
    gb = jnp.clip(jnp.mean(g, axis=0, keepdims=True), EPS, None)
    gbar_ref[...] = gb
    entg_ref[...] = jnp.sum(gb * jnp.log(gb), axis=1, keepdims=True)
    pc = jnp.clip(g, EPS, None)
    ent = jnp.sum(pc * jnp.log(pc), axis=1, keepdims=True)
    e = g.shape[1]
    tab_ref[...] = jnp.concatenate(
        [pc, jnp.broadcast_to(ent, (g.shape[0], e))], axis=1)


def _gbar(gates):
    n, e = gates.shape
    return pl.pallas_call(
        _gbar_body,
        out_shape=(jax.ShapeDtypeStruct((1, e), jnp.float32),
                   jax.ShapeDtypeStruct((1, 1), jnp.float32),
                   jax.ShapeDtypeStruct((n, 2 * e), jnp.float32)),
    )(gates)


def _topk_body(bm, n, z_ref, val_ref, idx_ref, zn_ref, sim_ref):
    i = pl.program_id(0)

    @pl.when(i == 0)
    def _():
        z = z_ref[...]
        nrm = jnp.sqrt(jnp.sum(z * z, axis=1, keepdims=True))
        zn_ref[...] = z / jnp.maximum(nrm, 1e-12)

    zl = zn_ref[pl.ds(i * bm, bm), :]
    sim_ref[...] = lax.dot_general(zl, zn_ref[...], (((1,), (1,)), ((), ())),
                                   preferred_element_type=jnp.float32)
    sub = sim_ref[:, pl.ds(i * bm, bm)]
    rloc = lax.broadcasted_iota(jnp.int32, (bm, bm), 0)
    cloc = lax.broadcasted_iota(jnp.int32, (bm, bm), 1)
    sim_ref[:, pl.ds(i * bm, bm)] = jnp.where(rloc == cloc, 0.0, sub)

    half = n // 2
    w = sim_ref[...]
    a = w[:, :half]
    b = w[:, half:]
    colf = lax.broadcasted_iota(jnp.int32, (1, half), 1).astype(jnp.float32)
    mx = jnp.maximum(a, b)
    mn = jnp.minimum(a, b)
    pmx = jnp.where(a >= b, colf, colf + half)
    m = jnp.max(mx, axis=1, keepdims=True)
    mvals, mpos, mpart = [], [], []
    for _ in range(K):
        c = mx == m
        mpos.append(jnp.min(jnp.where(c, pmx, 2e9), axis=1, keepdims=True))
        mpart.append(jnp.max(jnp.where(c, mn, NEG), axis=1, keepdims=True))
        mvals.append(m)
        mx = jnp.where(c, NEG, mx)
        m = jnp.max(mx, axis=1, keepdims=True)
    pp = jnp.concatenate(mpos, axis=1)
    ppart = jnp.where(pp < half, pp + half, pp - half)
    cv = jnp.concatenate(mvals + mpart, axis=1)
    ci = jnp.concatenate([pp, ppart], axis=1)
    colf16 = lax.broadcasted_iota(jnp.int32, (1, 2 * K), 1).astype(jnp.float32)
    vals, idxs = [], []
    for _ in range(K):
        m2 = jnp.max(cv, axis=1, keepdims=True)
        s = jnp.min(jnp.where(cv == m2, colf16, 2e9), axis=1, keepdims=True)
        sel = colf16 == s
        vals.append(m2)
        idxs.append(jnp.sum(jnp.where(sel, ci, 0.0), axis=1, keepdims=True))
        cv = jnp.where(sel, NEG, cv)
    val_ref[...] = jnp.concatenate(vals, axis=1)
    idx_ref[...] = jnp.concatenate(idxs, axis=1).astype(jnp.int32)


def _knn_top8(z):
    n, d = z.shape
    bm = 128
    return pl.pallas_call(
        functools.partial(_topk_body, bm, n),
        grid=(n // bm,),
        in_specs=[pl.BlockSpec((n, d), lambda i: (0, 0))],
        out_specs=(pl.BlockSpec((bm, K), lambda i: (i, 0)),
                   pl.BlockSpec((bm, K), lambda i: (i, 0))),
        out_shape=(jax.ShapeDtypeStruct((n, K), jnp.float32),
                   jax.ShapeDtypeStruct((n, K), jnp.int32)),
        scratch_shapes=[pltpu.VMEM((n, d), jnp.float32),
                        pltpu.VMEM((bm, n), jnp.float32)],
        compiler_params=pltpu.CompilerParams(
            dimension_semantics=("arbitrary",)),
    )(z)


def _gather_rows(table, idx):
    b = idx.shape[0]
    v, e = table.shape
    nw = 32
    chunk = 128
    b_per_w = b // nw
    nch = b_per_w // chunk
    mesh = plsc.VectorSubcoreMesh(core_axis_name="c", subcore_axis_name="s")

    @functools.partial(
        pl.kernel, mesh=mesh,
        out_type=jax.ShapeDtypeStruct((b, e), jnp.float32),
        scratch_types=[pltpu.VMEM((chunk,), jnp.int32),
                       pltpu.VMEM((chunk, e), jnp.float32),
                       pltpu.SemaphoreType.DMA],
    )
    def gather_k(table_hbm, idx_hbm, out_hbm, idx_v, rows_v, sem):
        wid = lax.axis_index("s") * 2 + lax.axis_index("c")
        base = wid * b_per_w
        for c in range(nch):
            off = base + c * chunk
            pltpu.sync_copy(idx_hbm.at[pl.ds(off, chunk)], idx_v)
            pltpu.async_copy(table_hbm.at[idx_v], rows_v, sem).wait()
            pltpu.sync_copy(rows_v, out_hbm.at[pl.ds(off, chunk)])

    return gather_k(table, idx)


def _loss_body(bm, n, e, t_ref, gj_ref, kv_ref, ti_ref, gbar_ref, entg_ref,
               out_ref, coh_s, ali_s, cnt_s):
    i = pl.program_id(0)
    ni = pl.num_programs(0)

    @pl.when(i == 0)
    def _():
        coh_s[...] = jnp.zeros((1, 1), jnp.float32)
        ali_s[...] = jnp.zeros((1, 1), jnp.float32)
        cnt_s[...] = jnp.zeros((1, e), jnp.float32)

    tab = t_ref[...]
    p = tab[:, :e]
    entp = tab[:, e:e + 1]
    gjt = gj_ref[...]
    q = gjt[:, :e]
    entq = gjt[:, e:e + 1].reshape(bm, K)
    s = p[:, None, :] + q.reshape(bm, K, e)
    slogs = jnp.sum(s * jnp.log(s), axis=2)
    ssum = jnp.sum(s, axis=2)
    js = 0.5 * (entp + entq - slogs + LOG2 * ssum)
    coh_s[...] += jnp.sum(kv_ref[...] * js, keepdims=True).reshape(1, 1)

    gb = gbar_ref[...]
    s2 = p + gb
    slogs2 = jnp.sum(s2 * jnp.log(s2), axis=1, keepdims=True)
    ssum2 = jnp.sum(s2, axis=1, keepdims=True)
    js2 = 0.5 * (entp + entg_ref[...] - slogs2 + LOG2 * ssum2)
    ali_s[...] += jnp.sum(js2, keepdims=True).reshape(1, 1)

    t0 = ti_ref[...][:, 0:1]
    eids = lax.broadcasted_iota(jnp.int32, (1, e), 1)
    cnt_s[...] += jnp.sum(jnp.where(t0 == eids, 1.0, 0.0), axis=0,
                          keepdims=True)

    @pl.when(i == ni - 1)
    def _():
        l_coh = coh_s[...] / (n * K)
        l_ali = ali_s[...] / n
        cnts = cnt_s[...]
        n_bar = jnp.maximum(jnp.sum(cnts, keepdims=True) / e, EPS)
        over = cnts / n_bar - TAU
        l_sep = jnp.sum(jnp.maximum(over, 0.0) ** 2, keepdims=True) / e
        loss = LC * l_coh + LS * l_sep + LA * l_ali
        out_ref[...] = jnp.concatenate([l_coh, l_sep, l_ali, loss], axis=1)


def _losses(table, gj, knn_val, topk_idx, gbar, entg):
    n = table.shape[0]
    e = table.shape[1] // 2
    bm = 512
    return pl.pallas_call(
        functools.partial(_loss_body, bm, n, e),
        grid=(n // bm,),
        in_specs=[pl.BlockSpec((bm, 2 * e), lambda i: (i, 0)),
                  pl.BlockSpec((bm * K, 2 * e), lambda i: (i, 0)),
                  pl.BlockSpec((bm, K), lambda i: (i, 0)),
                  pl.BlockSpec((bm, K), lambda i: (i, 0)),
                  pl.BlockSpec((1, e), lambda i: (0, 0)),
                  pl.BlockSpec((1, 1), lambda i: (0, 0))],
        out_specs=pl.BlockSpec((1, 4), lambda i: (0, 0)),
        out_shape=jax.ShapeDtypeStruct((1, 4), jnp.float32),
        scratch_shapes=[pltpu.VMEM((1, 1), jnp.float32),
                        pltpu.VMEM((1, 1), jnp.float32),
                        pltpu.VMEM((1, e), jnp.float32)],
        compiler_params=pltpu.CompilerParams(
            dimension_semantics=("arbitrary",)),
    )(table, gj, knn_val, topk_idx, gbar, entg)


def kernel(z, gates_soft, topk_idx, num_experts):
    del num_experts
    gbar, entg, table = _gbar(gates_soft)
    knn_val, knn_idx = _knn_top8(z)
    gj = _gather_rows(table, knn_idx.reshape(-1))
    out = _losses(table, gj, knn_val, topk_idx, gbar, entg)
    return out.reshape(4)


# speedup vs baseline: 1.0410x; 1.0410x over previous
import functools

import jax
import jax.numpy as jnp
from jax import lax
from jax.experimental import pallas as pl
from jax.experimental.pallas import tpu as pltpu
from jax.experimental.pallas import tpu_sc as plsc

EPS = 1e-8
TAU = 1.5
LC, LS, LA = 0.1, 0.05, 0.01
K = 8
NEG = -1e30
BIG = 2 ** 30
LOG2 = 0.6931471805599453

def _gbar_body(g_ref, gbar_ref, entg_ref, tab_ref):
    g = g_ref[...]
    gb = jnp.clip(jnp.mean(g, axis=0, keepdims=True), EPS, None)
    gbar_ref[...] = gb
    entg_ref[...] = jnp.sum(gb * jnp.log(gb), axis=1, keepdims=True)
    pc = jnp.clip(g, EPS, None)
    ent = jnp.sum(pc * jnp.log(pc), axis=1, keepdims=True)
    e = g.shape[1]
    tab_ref[...] = jnp.concatenate(
        [pc, jnp.broadcast_to(ent, (g.shape[0], e))], axis=1)


def _gbar(gates):
    n, e = gates.shape
    return pl.pallas_call(
        _gbar_body,
        out_shape=(jax.ShapeDtypeStruct((1, e), jnp.float32),
                   jax.ShapeDtypeStruct((1, 1), jnp.float32),
                   jax.ShapeDtypeStruct((n, 2 * e), jnp.float32)),
    )(gates)


def _topk_body(bm, n, z_ref, val_ref, idx_ref, zn_ref, sim_ref):
    i = pl.program_id(0)

    @pl.when(i == 0)
    def _():
        z = z_ref[...]
        nrm = jnp.sqrt(jnp.sum(z * z, axis=1, keepdims=True))
        zn_ref[...] = z / jnp.maximum(nrm, 1e-12)

    zl = zn_ref[pl.ds(i * bm, bm), :]
    sim_ref[...] = lax.dot_general(zl, zn_ref[...], (((1,), (1,)), ((), ())),
                                   preferred_element_type=jnp.float32)
    sub = sim_ref[:, pl.ds(i * bm, bm)]
    rloc = lax.broadcasted_iota(jnp.int32, (bm, bm), 0)
    cloc = lax.broadcasted_iota(jnp.int32, (bm, bm), 1)
    sim_ref[:, pl.ds(i * bm, bm)] = jnp.where(rloc == cloc, 0.0, sub)

    half = n // 2
    w = sim_ref[...]
    a = w[:, :half]
    b = w[:, half:]
    colf = lax.broadcasted_iota(jnp.int32, (1, half), 1).astype(jnp.float32)
    mx = jnp.maximum(a, b)
    mn = jnp.minimum(a, b)
    pmx = jnp.where(a >= b, colf, colf + half)
    m = jnp.max(mx, axis=1, keepdims=True)
    mvals, mpos, mpart = [], [], []
    for _ in range(K):
        c = mx == m
        mpos.append(jnp.min(jnp.where(c, pmx, 2e9), axis=1, keepdims=True))
        mpart.append(jnp.max(jnp.where(c, mn, NEG), axis=1, keepdims=True))
        mvals.append(m)
        mx = jnp.where(c, NEG, mx)
        m = jnp.max(mx, axis=1, keepdims=True)
    pp = jnp.concatenate(mpos, axis=1)
    ppart = jnp.where(pp < half, pp + half, pp - half)
    cv = jnp.concatenate(mvals + mpart, axis=1)
    ci = jnp.concatenate([pp, ppart], axis=1)
    colf16 = lax.broadcasted_iota(jnp.int32, (1, 2 * K), 1).astype(jnp.float32)
    vals, idxs = [], []
    for _ in range(K):
        m2 = jnp.max(cv, axis=1, keepdims=True)
        s = jnp.min(jnp.where(cv == m2, colf16, 2e9), axis=1, keepdims=True)
        sel = colf16 == s
        vals.append(m2)
        idxs.append(jnp.sum(jnp.where(sel, ci, 0.0), axis=1, keepdims=True))
        cv = jnp.where(sel, NEG, cv)
    val_ref[...] = jnp.concatenate(vals, axis=1)
    idx_ref[...] = jnp.concatenate(idxs, axis=1).astype(jnp.int32)


def _knn_top8(z):
    n, d = z.shape
    bm = 256
    return pl.pallas_call(
        functools.partial(_topk_body, bm, n),
        grid=(n // bm,),
        in_specs=[pl.BlockSpec((n, d), lambda i: (0, 0))],
        out_specs=(pl.BlockSpec((bm, K), lambda i: (i, 0)),
                   pl.BlockSpec((bm, K), lambda i: (i, 0))),
        out_shape=(jax.ShapeDtypeStruct((n, K), jnp.float32),
                   jax.ShapeDtypeStruct((n, K), jnp.int32)),
        scratch_shapes=[pltpu.VMEM((n, d), jnp.float32),
                        pltpu.VMEM((bm, n), jnp.float32)],
        compiler_params=pltpu.CompilerParams(
            dimension_semantics=("arbitrary",)),
    )(z)


def _gather_rows(table, idx):
    b = idx.shape[0]
    v, e = table.shape
    nw = 32
    chunk = 128
    b_per_w = b // nw
    nch = b_per_w // chunk
    mesh = plsc.VectorSubcoreMesh(core_axis_name="c", subcore_axis_name="s")

    @functools.partial(
        pl.kernel, mesh=mesh,
        out_type=jax.ShapeDtypeStruct((b, e), jnp.float32),
        scratch_types=[pltpu.VMEM((chunk,), jnp.int32),
                       pltpu.VMEM((chunk, e), jnp.float32),
                       pltpu.SemaphoreType.DMA],
    )
    def gather_k(table_hbm, idx_hbm, out_hbm, idx_v, rows_v, sem):
        wid = lax.axis_index("s") * 2 + lax.axis_index("c")
        base = wid * b_per_w
        for c in range(nch):
            off = base + c * chunk
            pltpu.sync_copy(idx_hbm.at[pl.ds(off, chunk)], idx_v)
            pltpu.async_copy(table_hbm.at[idx_v], rows_v, sem).wait()
            pltpu.sync_copy(rows_v, out_hbm.at[pl.ds(off, chunk)])

    return gather_k(table, idx)


def _loss_body(bm, n, e, t_ref, gj_ref, kv_ref, ti_ref, gbar_ref, entg_ref,
               out_ref, coh_s, ali_s, cnt_s):
    i = pl.program_id(0)
    ni = pl.num_programs(0)

    @pl.when(i == 0)
    def _():
        coh_s[...] = jnp.zeros((1, 1), jnp.float32)
        ali_s[...] = jnp.zeros((1, 1), jnp.float32)
        cnt_s[...] = jnp.zeros((1, e), jnp.float32)

    tab = t_ref[...]
    p = tab[:, :e]
    entp = tab[:, e:e + 1]
    gjt = gj_ref[...]
    q = gjt[:, :e]
    entq = gjt[:, e:e + 1].reshape(bm, K)
    s = p[:, None, :] + q.reshape(bm, K, e)
    slogs = jnp.sum(s * jnp.log(s), axis=2)
    ssum = jnp.sum(s, axis=2)
    js = 0.5 * (entp + entq - slogs + LOG2 * ssum)
    coh_s[...] += jnp.sum(kv_ref[...] * js, keepdims=True).reshape(1, 1)

    gb = gbar_ref[...]
    s2 = p + gb
    slogs2 = jnp.sum(s2 * jnp.log(s2), axis=1, keepdims=True)
    ssum2 = jnp.sum(s2, axis=1, keepdims=True)
    js2 = 0.5 * (entp + entg_ref[...] - slogs2 + LOG2 * ssum2)
    ali_s[...] += jnp.sum(js2, keepdims=True).reshape(1, 1)

    t0 = ti_ref[...][:, 0:1]
    eids = lax.broadcasted_iota(jnp.int32, (1, e), 1)
    cnt_s[...] += jnp.sum(jnp.where(t0 == eids, 1.0, 0.0), axis=0,
                          keepdims=True)

    @pl.when(i == ni - 1)
    def _():
        l_coh = coh_s[...] / (n * K)
        l_ali = ali_s[...] / n
        cnts = cnt_s[...]
        n_bar = jnp.maximum(jnp.sum(cnts, keepdims=True) / e, EPS)
        over = cnts / n_bar - TAU
        l_sep = jnp.sum(jnp.maximum(over, 0.0) ** 2, keepdims=True) / e
        loss = LC * l_coh + LS * l_sep + LA * l_ali
        out_ref[...] = jnp.concatenate([l_coh, l_sep, l_ali, loss], axis=1)


def _losses(table, gj, knn_val, topk_idx, gbar, entg):
    n = table.shape[0]
    e = table.shape[1] // 2
    bm = 512
    return pl.pallas_call(
        functools.partial(_loss_body, bm, n, e),
        grid=(n // bm,),
        in_specs=[pl.BlockSpec((bm, 2 * e), lambda i: (i, 0)),
                  pl.BlockSpec((bm * K, 2 * e), lambda i: (i, 0)),
                  pl.BlockSpec((bm, K), lambda i: (i, 0)),
                  pl.BlockSpec((bm, K), lambda i: (i, 0)),
                  pl.BlockSpec((1, e), lambda i: (0, 0)),
                  pl.BlockSpec((1, 1), lambda i: (0, 0))],
        out_specs=pl.BlockSpec((1, 4), lambda i: (0, 0)),
        out_shape=jax.ShapeDtypeStruct((1, 4), jnp.float32),
        scratch_shapes=[pltpu.VMEM((1, 1), jnp.float32),
                        pltpu.VMEM((1, 1), jnp.float32),
                        pltpu.VMEM((1, e), jnp.float32)],
        compiler_params=pltpu.CompilerParams(
            dimension_semantics=("arbitrary",)),
    )(table, gj, knn_val, topk_idx, gbar, entg)


def kernel(z, gates_soft, topk_idx, num_experts):
    del num_experts
    gbar, entg, table = _gbar(gates_soft)
    knn_val, knn_idx = _knn_top8(z)
    gj = _gather_rows(table, knn_idx.reshape(-1))
    out = _losses(table, gj, knn_val, topk_idx, gbar, entg)
    return out.reshape(4)


# speedup vs baseline: 1.0747x; 1.0324x over previous
import functools

import jax
import jax.numpy as jnp
from jax import lax
from jax.experimental import pallas as pl
from jax.experimental.pallas import tpu as pltpu
from jax.experimental.pallas import tpu_sc as plsc

EPS = 1e-8
TAU = 1.5
LC, LS, LA = 0.1, 0.05, 0.01
K = 8
NEG = -1e30
BIG = 2 ** 30
LOG2 = 0.6931471805599453

def _gbar_body(g_ref, gbar_ref, entg_ref, tab_ref):
    g = g_ref[...]
    gb = jnp.clip(jnp.mean(g, axis=0, keepdims=True), EPS, None)
    gbar_ref[...] = gb
    entg_ref[...] = jnp.sum(gb * jnp.log(gb), axis=1, keepdims=True)
    pc = jnp.clip(g, EPS, None)
    ent = jnp.sum(pc * jnp.log(pc), axis=1, keepdims=True)
    e = g.shape[1]
    tab_ref[...] = jnp.concatenate(
        [pc, jnp.broadcast_to(ent, (g.shape[0], e))], axis=1)


def _gbar(gates):
    n, e = gates.shape
    return pl.pallas_call(
        _gbar_body,
        out_shape=(jax.ShapeDtypeStruct((1, e), jnp.float32),
                   jax.ShapeDtypeStruct((1, 1), jnp.float32),
                   jax.ShapeDtypeStruct((n, 2 * e), jnp.float32)),
    )(gates)


def _topk_body(bm, n, z_ref, val_ref, idx_ref, zn_ref, sim_ref):
    i = pl.program_id(0)

    @pl.when(i == 0)
    def _():
        z = z_ref[...]
        nrm = jnp.sqrt(jnp.sum(z * z, axis=1, keepdims=True))
        zn_ref[...] = z / jnp.maximum(nrm, 1e-12)

    zl = zn_ref[pl.ds(i * bm, bm), :]
    sim_ref[...] = lax.dot_general(zl, zn_ref[...], (((1,), (1,)), ((), ())),
                                   preferred_element_type=jnp.float32)
    sub = sim_ref[:, pl.ds(i * bm, bm)]
    rloc = lax.broadcasted_iota(jnp.int32, (bm, bm), 0)
    cloc = lax.broadcasted_iota(jnp.int32, (bm, bm), 1)
    sim_ref[:, pl.ds(i * bm, bm)] = jnp.where(rloc == cloc, 0.0, sub)

    half = n // 2
    w = sim_ref[...]
    a = w[:, :half]
    b = w[:, half:]
    colf = lax.broadcasted_iota(jnp.int32, (1, half), 1).astype(jnp.float32)
    mx = jnp.maximum(a, b)
    mn = jnp.minimum(a, b)
    pmx = jnp.where(a >= b, colf, colf + half)
    m = jnp.max(mx, axis=1, keepdims=True)
    mvals, mpos, mpart = [], [], []
    for _ in range(K):
        c = mx == m
        mpos.append(jnp.min(jnp.where(c, pmx, 2e9), axis=1, keepdims=True))
        mpart.append(jnp.max(jnp.where(c, mn, NEG), axis=1, keepdims=True))
        mvals.append(m)
        mx = jnp.where(c, NEG, mx)
        m = jnp.max(mx, axis=1, keepdims=True)
    pp = jnp.concatenate(mpos, axis=1)
    ppart = jnp.where(pp < half, pp + half, pp - half)
    cv = jnp.concatenate(mvals + mpart, axis=1)
    ci = jnp.concatenate([pp, ppart], axis=1)
    colf16 = lax.broadcasted_iota(jnp.int32, (1, 2 * K), 1).astype(jnp.float32)
    vals, idxs = [], []
    for _ in range(K):
        m2 = jnp.max(cv, axis=1, keepdims=True)
        s = jnp.min(jnp.where(cv == m2, colf16, 2e9), axis=1, keepdims=True)
        sel = colf16 == s
        vals.append(m2)
        idxs.append(jnp.sum(jnp.where(sel, ci, 0.0), axis=1, keepdims=True))
        cv = jnp.where(sel, NEG, cv)
    val_ref[...] = jnp.concatenate(vals, axis=1)
    idx_ref[...] = jnp.concatenate(idxs, axis=1).astype(jnp.int32)


def _knn_top8(z):
    n, d = z.shape
    bm = 256
    return pl.pallas_call(
        functools.partial(_topk_body, bm, n),
        grid=(n // bm,),
        in_specs=[pl.BlockSpec((n, d), lambda i: (0, 0))],
        out_specs=(pl.BlockSpec((bm, K), lambda i: (i, 0)),
                   pl.BlockSpec((bm, K), lambda i: (i, 0))),
        out_shape=(jax.ShapeDtypeStruct((n, K), jnp.float32),
                   jax.ShapeDtypeStruct((n, K), jnp.int32)),
        scratch_shapes=[pltpu.VMEM((n, d), jnp.float32),
                        pltpu.VMEM((bm, n), jnp.float32)],
        compiler_params=pltpu.CompilerParams(
            dimension_semantics=("arbitrary",)),
    )(z)


def _gather_rows(table, idx):
    b = idx.shape[0]
    v, e = table.shape
    nw = 32
    chunk = 128
    b_per_w = b // nw
    nch = b_per_w // chunk
    mesh = plsc.VectorSubcoreMesh(core_axis_name="c", subcore_axis_name="s")

    @functools.partial(
        pl.kernel, mesh=mesh,
        out_type=jax.ShapeDtypeStruct((b, e), jnp.float32),
        scratch_types=[pltpu.VMEM((chunk,), jnp.int32),
                       pltpu.VMEM((chunk,), jnp.int32),
                       pltpu.VMEM((chunk, e), jnp.float32),
                       pltpu.VMEM((chunk, e), jnp.float32),
                       pltpu.SemaphoreType.DMA,
                       pltpu.SemaphoreType.DMA],
    )
    def gather_k(table_hbm, idx_hbm, out_hbm, idx0, idx1, rows0, rows1,
                 sem0, sem1):
        wid = lax.axis_index("s") * 2 + lax.axis_index("c")
        base = wid * b_per_w
        idx_v = (idx0, idx1)
        rows_v = (rows0, rows1)
        sems = (sem0, sem1)
        pltpu.sync_copy(idx_hbm.at[pl.ds(base, chunk)], idx0)
        pend = [pltpu.async_copy(table_hbm.at[idx0], rows0, sem0)]
        for c in range(nch):
            if c + 1 < nch:
                nb = (c + 1) % 2
                pltpu.sync_copy(
                    idx_hbm.at[pl.ds(base + (c + 1) * chunk, chunk)],
                    idx_v[nb])
                pend.append(pltpu.async_copy(table_hbm.at[idx_v[nb]],
                                             rows_v[nb], sems[nb]))
            pend.pop(0).wait()
            pltpu.sync_copy(rows_v[c % 2],
                            out_hbm.at[pl.ds(base + c * chunk, chunk)])

    return gather_k(table, idx)


def _loss_body(bm, n, e, t_ref, gj_ref, kv_ref, ti_ref, gbar_ref, entg_ref,
               out_ref, coh_s, ali_s, cnt_s):
    i = pl.program_id(0)
    ni = pl.num_programs(0)

    @pl.when(i == 0)
    def _():
        coh_s[...] = jnp.zeros((1, 1), jnp.float32)
        ali_s[...] = jnp.zeros((1, 1), jnp.float32)
        cnt_s[...] = jnp.zeros((1, e), jnp.float32)

    tab = t_ref[...]
    p = tab[:, :e]
    entp = tab[:, e:e + 1]
    gjt = gj_ref[...]
    q = gjt[:, :e]
    entq = gjt[:, e:e + 1].reshape(bm, K)
    s = p[:, None, :] + q.reshape(bm, K, e)
    slogs = jnp.sum(s * jnp.log(s), axis=2)
    ssum = jnp.sum(s, axis=2)
    js = 0.5 * (entp + entq - slogs + LOG2 * ssum)
    coh_s[...] += jnp.sum(kv_ref[...] * js, keepdims=True).reshape(1, 1)

    gb = gbar_ref[...]
    s2 = p + gb
    slogs2 = jnp.sum(s2 * jnp.log(s2), axis=1, keepdims=True)
    ssum2 = jnp.sum(s2, axis=1, keepdims=True)
    js2 = 0.5 * (entp + entg_ref[...] - slogs2 + LOG2 * ssum2)
    ali_s[...] += jnp.sum(js2, keepdims=True).reshape(1, 1)

    t0 = ti_ref[...][:, 0:1]
    eids = lax.broadcasted_iota(jnp.int32, (1, e), 1)
    cnt_s[...] += jnp.sum(jnp.where(t0 == eids, 1.0, 0.0), axis=0,
                          keepdims=True)

    @pl.when(i == ni - 1)
    def _():
        l_coh = coh_s[...] / (n * K)
        l_ali = ali_s[...] / n
        cnts = cnt_s[...]
        n_bar = jnp.maximum(jnp.sum(cnts, keepdims=True) / e, EPS)
        over = cnts / n_bar - TAU
        l_sep = jnp.sum(jnp.maximum(over, 0.0) ** 2, keepdims=True) / e
        loss = LC * l_coh + LS * l_sep + LA * l_ali
        out_ref[...] = jnp.concatenate([l_coh, l_sep, l_ali, loss], axis=1)


def _losses(table, gj, knn_val, topk_idx, gbar, entg):
    n = table.shape[0]
    e = table.shape[1] // 2
    bm = 1024
    return pl.pallas_call(
        functools.partial(_loss_body, bm, n, e),
        grid=(n // bm,),
        in_specs=[pl.BlockSpec((bm, 2 * e), lambda i: (i, 0)),
                  pl.BlockSpec((bm * K, 2 * e), lambda i: (i, 0)),
                  pl.BlockSpec((bm, K), lambda i: (i, 0)),
                  pl.BlockSpec((bm, K), lambda i: (i, 0)),
                  pl.BlockSpec((1, e), lambda i: (0, 0)),
                  pl.BlockSpec((1, 1), lambda i: (0, 0))],
        out_specs=pl.BlockSpec((1, 4), lambda i: (0, 0)),
        out_shape=jax.ShapeDtypeStruct((1, 4), jnp.float32),
        scratch_shapes=[pltpu.VMEM((1, 1), jnp.float32),
                        pltpu.VMEM((1, 1), jnp.float32),
                        pltpu.VMEM((1, e), jnp.float32)],
        compiler_params=pltpu.CompilerParams(
            dimension_semantics=("arbitrary",)),
    )(table, gj, knn_val, topk_idx, gbar, entg)


def kernel(z, gates_soft, topk_idx, num_experts):
    del num_experts
    gbar, entg, table = _gbar(gates_soft)
    knn_val, knn_idx = _knn_top8(z)
    gj = _gather_rows(table, knn_idx.reshape(-1))
    out = _losses(table, gj, knn_val, topk_idx, gbar, entg)
    return out.reshape(4)
